# baseline (speedup 1.0000x reference)
import functools

import jax
import jax.numpy as jnp
from jax import lax
from jax.experimental import pallas as pl
from jax.experimental.pallas import tpu as pltpu
from jax.experimental.pallas import tpu_sc as plsc

_D_IN = 768
_D_SAE = 12288
_K = 32
_N_TOK = 2048
_TOK_BLK_A = 128
_TOK_BLK_B = 64

_NW = 32
_RPW = _N_TOK // _NW
_G = 4
_NG = _RPW // _G
_NCH = _D_SAE // 16


def _encode_body(x_ref, we_ref, be_ref, bd_ref, feats_ref):
    x = x_ref[...]
    hidden = jnp.dot(x - bd_ref[...], we_ref[...],
                     preferred_element_type=jnp.float32)
    feats_ref[...] = jnp.maximum(hidden + be_ref[...], 0.0)


def _sort_d(v):
    return plsc.sort_key_val(v, v, descending=True)[0]


def _sort_a(v):
    return plsc.sort_key_val(v, v, descending=False)[0]


def _sc_body(feats_hbm, out_hbm, buf0, buf1, tbuf, sem0, sem1):
    nc = 2
    wid = lax.axis_index("s") * nc + lax.axis_index("c")
    base = wid * _RPW

    _H = 4
    _HCH = _NCH // _H

    def _feed(b1, b2, s_desc):
        nb = jnp.maximum(b1, s_desc)
        lo = jnp.minimum(b1, s_desc)
        s2 = _sort_d(lo)
        return _sort_a(nb), _sort_a(jnp.maximum(b2, s2))

    def _process(buf, g):
        neg = jnp.full((16,), -jnp.inf, jnp.float32)
        st = []
        for r in range(_G):
            for h in range(_H):
                st += [_sort_a(buf[r, pl.ds(h * _HCH * 16, 16)]), neg]

        def jbody(j, carry):
            out = []
            for r in range(_G):
                for h in range(_H):
                    i0 = 2 * (r * _H + h)
                    off = pl.multiple_of(j * 16 + h * (_HCH * 16), 16)
                    s = _sort_d(buf[r, pl.ds(off, 16)])
                    b1, b2 = _feed(carry[i0], carry[i0 + 1], s)
                    out += [b1, b2]
            return tuple(out)

        st = lax.fori_loop(1, _HCH, jbody, tuple(st))

        for r in range(_G):
            i0 = 2 * (r * _H)
            b1, b2 = st[i0], st[i0 + 1]
            for h in range(1, _H):
                ih = 2 * (r * _H + h)
                for vsrc in (st[ih], st[ih + 1]):
                    b1, b2 = _feed(b1, b2, _sort_d(vsrc))
            t = jnp.min(b2)
            tv = jnp.broadcast_to(t, (16,))
            for c in range(8):
                tbuf[r, pl.ds(c * 16, 16)] = tv
        pltpu.sync_copy(tbuf, out_hbm.at[pl.ds(base + g * _G, _G)])

    bufs = (buf0, buf1)
    sems = (sem0, sem1)
    pending = pltpu.async_copy(feats_hbm.at[pl.ds(base, _G)], buf0, sem0)
    for g in range(_NG):
        cur = bufs[g % 2]
        if g + 1 < _NG:
            nxt = pltpu.async_copy(
                feats_hbm.at[pl.ds(base + (g + 1) * _G, _G)],
                bufs[(g + 1) % 2], sems[(g + 1) % 2])
        pending.wait()
        _process(cur, g)
        if g + 1 < _NG:
            pending = nxt


_sc_thresh = functools.partial(
    pl.kernel,
    out_type=jax.ShapeDtypeStruct((_N_TOK, 128), jnp.float32),
    mesh=plsc.VectorSubcoreMesh(core_axis_name="c", subcore_axis_name="s"),
    scratch_types=[
        pltpu.VMEM((_G, _D_SAE), jnp.float32),
        pltpu.VMEM((_G, _D_SAE), jnp.float32),
        pltpu.VMEM((_G, 128), jnp.float32),
        pltpu.SemaphoreType.DMA,
        pltpu.SemaphoreType.DMA,
    ],
    compiler_params=pltpu.CompilerParams(needs_layout_passes=False),
)(_sc_body)


def _decode_body(feats_ref, t_ref, wd_ref, bd_ref, x_ref,
                 sparse_ref, out_ref, fvu_ref,
                 err_acc, xs_acc, xq_acc, *, n_tok):
    i = pl.program_id(0)
    nsteps = pl.num_programs(0)
    b = feats_ref.shape[0]

    feats = feats_ref[...]
    t128 = t_ref[...]
    f3 = feats.reshape(b, _D_SAE // 128, 128)
    mask = (f3 >= t128[:, None, :]) & (f3 > 0.0)
    sparse = jnp.where(mask, f3, 0.0).reshape(b, _D_SAE)
    sparse_ref[...] = sparse

    x = x_ref[...]
    sae_out = jnp.dot(sparse, wd_ref[...],
                      preferred_element_type=jnp.float32) + bd_ref[...]
    out_ref[...] = sae_out

    err = sae_out - x
    e2 = jnp.sum(err * err, axis=0, keepdims=True)
    xs = jnp.sum(x, axis=0, keepdims=True)
    xq = jnp.sum(x * x, axis=0, keepdims=True)

    @pl.when(i == 0)
    def _():
        err_acc[...] = e2
        xs_acc[...] = xs
        xq_acc[...] = xq

    @pl.when(i > 0)
    def _():
        err_acc[...] += e2
        xs_acc[...] += xs
        xq_acc[...] += xq

    @pl.when(i == nsteps - 1)
    def _():
        xs_tot = xs_acc[...]
        tot_var = xq_acc[...] - xs_tot * xs_tot * (1.0 / n_tok)
        fvu_ref[...] = jnp.mean(err_acc[...] / tot_var).reshape(1, 1)


def kernel(x, W_enc, b_enc, W_dec, b_dec):
    n_tok = x.shape[0]
    be2 = b_enc.reshape(1, _D_SAE)
    bd2 = b_dec.reshape(1, _D_IN)

    feats = pl.pallas_call(
        _encode_body,
        grid=(n_tok // _TOK_BLK_A,),
        in_specs=[
            pl.BlockSpec((_TOK_BLK_A, _D_IN), lambda i: (i, 0)),
            pl.BlockSpec((_D_IN, _D_SAE), lambda i: (0, 0)),
            pl.BlockSpec((1, _D_SAE), lambda i: (0, 0)),
            pl.BlockSpec((1, _D_IN), lambda i: (0, 0)),
        ],
        out_specs=pl.BlockSpec((_TOK_BLK_A, _D_SAE), lambda i: (i, 0)),
        out_shape=jax.ShapeDtypeStruct((n_tok, _D_SAE), jnp.float32),
        compiler_params=pltpu.CompilerParams(
            dimension_semantics=("arbitrary",),
        ),
    )(x, W_enc, be2, bd2)

    thr = _sc_thresh(feats)

    sparse, sae_out, fvu = pl.pallas_call(
        functools.partial(_decode_body, n_tok=n_tok),
        grid=(n_tok // _TOK_BLK_B,),
        in_specs=[
            pl.BlockSpec((_TOK_BLK_B, _D_SAE), lambda i: (i, 0)),
            pl.BlockSpec((_TOK_BLK_B, 128), lambda i: (i, 0)),
            pl.BlockSpec((_D_SAE, _D_IN), lambda i: (0, 0)),
            pl.BlockSpec((1, _D_IN), lambda i: (0, 0)),
            pl.BlockSpec((_TOK_BLK_B, _D_IN), lambda i: (i, 0)),
        ],
        out_specs=[
            pl.BlockSpec((_TOK_BLK_B, _D_SAE), lambda i: (i, 0)),
            pl.BlockSpec((_TOK_BLK_B, _D_IN), lambda i: (i, 0)),
            pl.BlockSpec((1, 1), lambda i: (0, 0)),
        ],
        out_shape=[
            jax.ShapeDtypeStruct((n_tok, _D_SAE), jnp.float32),
            jax.ShapeDtypeStruct((n_tok, _D_IN), jnp.float32),
            jax.ShapeDtypeStruct((1, 1), jnp.float32),
        ],
        scratch_shapes=[
            pltpu.VMEM((1, _D_IN), jnp.float32),
            pltpu.VMEM((1, _D_IN), jnp.float32),
            pltpu.VMEM((1, _D_IN), jnp.float32),
        ],
        compiler_params=pltpu.CompilerParams(
            dimension_semantics=("arbitrary",),
        ),
    )(feats, thr, W_dec, bd2, x)

    return sae_out, sparse, fvu[0, 0]


# speedup vs baseline: 1.0034x; 1.0034x over previous
import functools

import jax
import jax.numpy as jnp
from jax import lax
from jax.experimental import pallas as pl
from jax.experimental.pallas import tpu as pltpu
from jax.experimental.pallas import tpu_sc as plsc

_D_IN = 768
_D_SAE = 12288
_K = 32
_N_TOK = 2048
_TOK_BLK_A = 128
_TOK_BLK_B = 64

_NW = 32
_RPW = _N_TOK // _NW
_G = 4
_NG = _RPW // _G
_NCH = _D_SAE // 16


def _encode_body(x_ref, we_ref, be_ref, bd_ref, feats_ref):
    x = x_ref[...]
    hidden = jnp.dot(x - bd_ref[...], we_ref[...],
                     preferred_element_type=jnp.float32)
    feats_ref[...] = jnp.maximum(hidden + be_ref[...], 0.0)


def _sort_d(v):
    return plsc.sort_key_val(v, v, descending=True)[0]


def _sort_a(v):
    return plsc.sort_key_val(v, v, descending=False)[0]


def _sc_body(feats_hbm, out_hbm, buf0, buf1, tbuf, sem0, sem1):
    nc = 2
    wid = lax.axis_index("s") * nc + lax.axis_index("c")
    base = wid * _RPW

    _H = 2
    _HCH = _NCH // _H

    def _feed(b1, b2, s_desc):
        nb = jnp.maximum(b1, s_desc)
        lo = jnp.minimum(b1, s_desc)
        s2 = _sort_d(lo)
        return _sort_a(nb), _sort_a(jnp.maximum(b2, s2))

    def _process(buf, g):
        neg = jnp.full((16,), -jnp.inf, jnp.float32)
        st = []
        for r in range(_G):
            for h in range(_H):
                st += [_sort_a(buf[r, pl.ds(h * _HCH * 16, 16)]), neg]

        def jbody(j, carry):
            out = []
            for r in range(_G):
                for h in range(_H):
                    i0 = 2 * (r * _H + h)
                    off = pl.multiple_of(j * 16 + h * (_HCH * 16), 16)
                    s = _sort_d(buf[r, pl.ds(off, 16)])
                    b1, b2 = _feed(carry[i0], carry[i0 + 1], s)
                    out += [b1, b2]
            return tuple(out)

        st = lax.fori_loop(1, _HCH, jbody, tuple(st))

        for r in range(_G):
            i0 = 2 * (r * _H)
            b1, b2 = st[i0], st[i0 + 1]
            for h in range(1, _H):
                ih = 2 * (r * _H + h)
                for vsrc in (st[ih], st[ih + 1]):
                    b1, b2 = _feed(b1, b2, _sort_d(vsrc))
            t = jnp.min(b2)
            tv = jnp.broadcast_to(t, (16,))
            for c in range(8):
                tbuf[r, pl.ds(c * 16, 16)] = tv
        pltpu.sync_copy(tbuf, out_hbm.at[pl.ds(base + g * _G, _G)])

    bufs = (buf0, buf1)
    sems = (sem0, sem1)
    pending = pltpu.async_copy(feats_hbm.at[pl.ds(base, _G)], buf0, sem0)
    for g in range(_NG):
        cur = bufs[g % 2]
        if g + 1 < _NG:
            nxt = pltpu.async_copy(
                feats_hbm.at[pl.ds(base + (g + 1) * _G, _G)],
                bufs[(g + 1) % 2], sems[(g + 1) % 2])
        pending.wait()
        _process(cur, g)
        if g + 1 < _NG:
            pending = nxt


_sc_thresh = functools.partial(
    pl.kernel,
    out_type=jax.ShapeDtypeStruct((_N_TOK, 128), jnp.float32),
    mesh=plsc.VectorSubcoreMesh(core_axis_name="c", subcore_axis_name="s"),
    scratch_types=[
        pltpu.VMEM((_G, _D_SAE), jnp.float32),
        pltpu.VMEM((_G, _D_SAE), jnp.float32),
        pltpu.VMEM((_G, 128), jnp.float32),
        pltpu.SemaphoreType.DMA,
        pltpu.SemaphoreType.DMA,
    ],
    compiler_params=pltpu.CompilerParams(needs_layout_passes=False),
)(_sc_body)


def _decode_body(feats_ref, t_ref, wd_ref, bd_ref, x_ref,
                 sparse_ref, out_ref, fvu_ref,
                 err_acc, xs_acc, xq_acc, *, n_tok):
    i = pl.program_id(0)
    nsteps = pl.num_programs(0)
    b = feats_ref.shape[0]

    feats = feats_ref[...]
    t128 = t_ref[...]
    f3 = feats.reshape(b, _D_SAE // 128, 128)
    mask = (f3 >= t128[:, None, :]) & (f3 > 0.0)
    sparse = jnp.where(mask, f3, 0.0).reshape(b, _D_SAE)
    sparse_ref[...] = sparse

    x = x_ref[...]
    sae_out = jnp.dot(sparse, wd_ref[...],
                      preferred_element_type=jnp.float32) + bd_ref[...]
    out_ref[...] = sae_out

    err = sae_out - x
    e2 = jnp.sum(err * err, axis=0, keepdims=True)
    xs = jnp.sum(x, axis=0, keepdims=True)
    xq = jnp.sum(x * x, axis=0, keepdims=True)

    @pl.when(i == 0)
    def _():
        err_acc[...] = e2
        xs_acc[...] = xs
        xq_acc[...] = xq

    @pl.when(i > 0)
    def _():
        err_acc[...] += e2
        xs_acc[...] += xs
        xq_acc[...] += xq

    @pl.when(i == nsteps - 1)
    def _():
        xs_tot = xs_acc[...]
        tot_var = xq_acc[...] - xs_tot * xs_tot * (1.0 / n_tok)
        fvu_ref[...] = jnp.mean(err_acc[...] / tot_var).reshape(1, 1)


def kernel(x, W_enc, b_enc, W_dec, b_dec):
    n_tok = x.shape[0]
    be2 = b_enc.reshape(1, _D_SAE)
    bd2 = b_dec.reshape(1, _D_IN)

    feats = pl.pallas_call(
        _encode_body,
        grid=(n_tok // _TOK_BLK_A,),
        in_specs=[
            pl.BlockSpec((_TOK_BLK_A, _D_IN), lambda i: (i, 0)),
            pl.BlockSpec((_D_IN, _D_SAE), lambda i: (0, 0)),
            pl.BlockSpec((1, _D_SAE), lambda i: (0, 0)),
            pl.BlockSpec((1, _D_IN), lambda i: (0, 0)),
        ],
        out_specs=pl.BlockSpec((_TOK_BLK_A, _D_SAE), lambda i: (i, 0)),
        out_shape=jax.ShapeDtypeStruct((n_tok, _D_SAE), jnp.float32),
        compiler_params=pltpu.CompilerParams(
            dimension_semantics=("arbitrary",),
        ),
    )(x, W_enc, be2, bd2)

    thr = _sc_thresh(feats)

    sparse, sae_out, fvu = pl.pallas_call(
        functools.partial(_decode_body, n_tok=n_tok),
        grid=(n_tok // _TOK_BLK_B,),
        in_specs=[
            pl.BlockSpec((_TOK_BLK_B, _D_SAE), lambda i: (i, 0)),
            pl.BlockSpec((_TOK_BLK_B, 128), lambda i: (i, 0)),
            pl.BlockSpec((_D_SAE, _D_IN), lambda i: (0, 0)),
            pl.BlockSpec((1, _D_IN), lambda i: (0, 0)),
            pl.BlockSpec((_TOK_BLK_B, _D_IN), lambda i: (i, 0)),
        ],
        out_specs=[
            pl.BlockSpec((_TOK_BLK_B, _D_SAE), lambda i: (i, 0)),
            pl.BlockSpec((_TOK_BLK_B, _D_IN), lambda i: (i, 0)),
            pl.BlockSpec((1, 1), lambda i: (0, 0)),
        ],
        out_shape=[
            jax.ShapeDtypeStruct((n_tok, _D_SAE), jnp.float32),
            jax.ShapeDtypeStruct((n_tok, _D_IN), jnp.float32),
            jax.ShapeDtypeStruct((1, 1), jnp.float32),
        ],
        scratch_shapes=[
            pltpu.VMEM((1, _D_IN), jnp.float32),
            pltpu.VMEM((1, _D_IN), jnp.float32),
            pltpu.VMEM((1, _D_IN), jnp.float32),
        ],
        compiler_params=pltpu.CompilerParams(
            dimension_semantics=("arbitrary",),
        ),
    )(feats, thr, W_dec, bd2, x)

    return sae_out, sparse, fvu[0, 0]
